# baseline (speedup 1.0000x reference)
import functools

import jax
import jax.numpy as jnp
import numpy as np
from jax import lax
from jax.experimental import pallas as pl
from jax.experimental.pallas import tpu as pltpu
from jax.experimental.pallas import tpu_sc as plsc

N, E, D = 10000, 320000, 128

_NC, _NS, _L = 2, 16, 16
_NW = _NC * _NS
_CH = 64
_NCHUNKS = E // _CH
_CPW = 160
_CPAD = _NW * _CPW
_EPAD = _CPAD * _CH
_NBUF = 4
_NPAD = 10112
_RPT = _NPAD // _NS
_EBLK = 6400

_CLO = np.concatenate([np.arange(32 * g, 32 * g + 16) for g in range(D // 32)])
_CHI = _CLO + 16


def _edge_mlp_body(ea_ref, Wlo_ref, blo_ref, Whi_ref, bhi_ref, out_ref):
    dn = (((1,), (1,)), ((), ()))
    ea = ea_ref[...]
    a = lax.dot_general(ea, Wlo_ref[...], dn,
                        preferred_element_type=jnp.float32) + blo_ref[...]
    bb = lax.dot_general(ea, Whi_ref[...], dn,
                         preferred_element_type=jnp.float32) + bhi_ref[...]
    au = lax.bitcast_convert_type(a.astype(jnp.bfloat16),
                                  jnp.uint16).astype(jnp.uint32)
    bu = lax.bitcast_convert_type(bb.astype(jnp.bfloat16),
                                  jnp.uint16).astype(jnp.uint32)
    out_ref[...] = au | lax.shift_left(bu, jnp.uint32(16))


def _edge_mlp(edge_attr, W_e, b_e):
    clo = jnp.asarray(_CLO)
    chi = jnp.asarray(_CHI)
    return pl.pallas_call(
        _edge_mlp_body,
        grid=(E // _EBLK,),
        in_specs=[
            pl.BlockSpec((_EBLK, D), lambda i: (i, 0)),
            pl.BlockSpec((D // 2, D), lambda i: (0, 0)),
            pl.BlockSpec((1, D // 2), lambda i: (0, 0)),
            pl.BlockSpec((D // 2, D), lambda i: (0, 0)),
            pl.BlockSpec((1, D // 2), lambda i: (0, 0)),
        ],
        out_specs=pl.BlockSpec((_EBLK, D // 2), lambda i: (i, 0)),
        out_shape=jax.ShapeDtypeStruct((E, D // 2), jnp.uint32),
    )(edge_attr, W_e[clo], b_e[clo].reshape(1, D // 2),
      W_e[chi], b_e[chi].reshape(1, D // 2))


_STRIPE = [(k * _CH, _CH) for k in range(_RPT // _CH)]
if _RPT % _CH:
    _STRIPE.append(((_RPT // _CH) * _CH, _RPT % _CH))

def _mp_body(x_hbm, src_hbm, dst_hbm, e_hbm, out_hbm,
             idx_s, idx_d, msgb, msgf, aggr_sh, sem_e, sem_g, sem_i, sem_sc):
    cid = lax.axis_index("c")
    sid = lax.axis_index("s")
    wid = cid * _NS + sid
    c0 = wid * _CPW

    def zrow(r, carry):
        for j in range(D // _L):
            msgf[0][r, pl.ds(j * _L, _L)] = jnp.zeros((_L,), jnp.float32)
        return carry
    lax.fori_loop(0, _CH, zrow, 0)
    for off, nr in _STRIPE:
        r0 = pl.multiple_of(sid * _RPT + off, 8)
        pltpu.sync_copy(msgf[0].at[pl.ds(0, nr)], aggr_sh.at[pl.ds(r0, nr)])
    plsc.subcore_barrier()

    def _ebase(c):
        g = c0 + c
        g = jnp.where(g < _NCHUNKS, g, 0)
        return pl.multiple_of(g * _CH, _CH)

    def _ibase(c):
        return pl.multiple_of((c0 + c) * _CH, _CH)

    def _i_issue(b, c):
        base = _ibase(c)
        pltpu.async_copy(src_hbm.at[pl.ds(base, _CH)], idx_s[b], sem_i[b])
        pltpu.async_copy(dst_hbm.at[pl.ds(base, _CH)], idx_d[b], sem_i[b])

    def _i_wait(b, c):
        base = _ibase(c)
        pltpu.make_async_copy(src_hbm.at[pl.ds(base, _CH)], idx_s[b],
                              sem_i[b]).wait()
        pltpu.make_async_copy(dst_hbm.at[pl.ds(base, _CH)], idx_d[b],
                              sem_i[b]).wait()

    def _e_issue(b, c):
        pltpu.async_copy(e_hbm.at[pl.ds(_ebase(c), _CH)], msgb[b], sem_e[b])

    def _e_wait(b, c):
        pltpu.make_async_copy(e_hbm.at[pl.ds(_ebase(c), _CH)], msgb[b],
                              sem_e[b]).wait()

    def _g_issue(b):
        pltpu.async_copy(x_hbm.at[idx_s[b]], msgf[b], sem_g[b])

    def _g_wait(b):
        pltpu.make_async_copy(x_hbm.at[idx_s[b]], msgf[b], sem_g[b]).wait()

    def _sc_issue(b):
        pltpu.async_copy(msgf[b], aggr_sh.at[idx_d[b]], sem_sc[b], add=True)

    def _sc_wait(b):
        pltpu.make_async_copy(msgf[b], aggr_sh.at[idx_d[b]], sem_sc[b]).wait()

    for k in range(2):
        _i_issue(k, k)
        _e_issue(k, k)
    for k in range(2):
        _i_wait(k, k)
        _g_issue(k)

    def body(t, carry):
        for b in range(_NBUF):
            c = t * _NBUF + b
            b2 = (b + 2) % _NBUF
            be = b % 2
            _g_wait(b)
            _e_wait(be, c)

            @pl.when(c >= 2)
            def _():
                _sc_wait(b2)

            @pl.when(c + 2 < _CPW)
            def _():
                _i_issue(b2, c + 2)

            hi_mask = jnp.uint32(0xFFFF0000)
            sh16 = jnp.uint32(16)

            def rrow(r, c2):
                for j in range(D // (2 * _L)):
                    u = msgb[be][r, pl.ds(j * _L, _L)]
                    lo = plsc.bitcast(lax.shift_left(u, sh16), jnp.float32)
                    hi = plsc.bitcast(u & hi_mask, jnp.float32)
                    jf = 2 * _L * j
                    x0 = msgf[b][r, pl.ds(jf, _L)]
                    x1 = msgf[b][r, pl.ds(jf + _L, _L)]
                    msgf[b][r, pl.ds(jf, _L)] = jnp.maximum(x0 + lo, 0.0)
                    msgf[b][r, pl.ds(jf + _L, _L)] = jnp.maximum(x1 + hi, 0.0)
                return c2
            lax.fori_loop(0, _CH, rrow, 0)

            @pl.when(c + 2 < _CPW)
            def _():
                _e_issue(be, c + 2)
                _i_wait(b2, c + 2)
                _g_issue(b2)
            _sc_issue(b)
        return carry
    lax.fori_loop(0, _CPW // _NBUF, body, 0)
    _sc_wait((_CPW - 2) % _NBUF)
    _sc_wait((_CPW - 1) % _NBUF)

    plsc.subcore_barrier()
    for off, nr in _STRIPE:
        r0 = pl.multiple_of(sid * _RPT + off, 8)
        pltpu.sync_copy(aggr_sh.at[pl.ds(r0, nr)], msgf[0].at[pl.ds(0, nr)])
        pltpu.sync_copy(msgf[0].at[pl.ds(0, nr)], out_hbm.at[cid, pl.ds(r0, nr)])


def _message_passing(x_p, src, dst, e):
    mesh = plsc.VectorSubcoreMesh(core_axis_name="c", subcore_axis_name="s")
    f = functools.partial(
        pl.kernel,
        out_type=jax.ShapeDtypeStruct((_NC, _NPAD, D), jnp.float32),
        mesh=mesh,
        compiler_params=pltpu.CompilerParams(needs_layout_passes=False),
        scratch_types=[
            [pltpu.VMEM((_CH,), jnp.int32)] * _NBUF,
            [pltpu.VMEM((_CH,), jnp.int32)] * _NBUF,
            [pltpu.VMEM((_CH, D // 2), jnp.uint32)] * 2,
            [pltpu.VMEM((_CH, D), jnp.float32)] * _NBUF,
            pltpu.VMEM_SHARED((_NPAD, D), jnp.float32),
            [pltpu.SemaphoreType.DMA] * 2,
            [pltpu.SemaphoreType.DMA] * _NBUF,
            [pltpu.SemaphoreType.DMA] * _NBUF,
            [pltpu.SemaphoreType.DMA] * _NBUF,
        ],
    )(_mp_body)
    return f(x_p, src, dst, e)


def _final_body(x_ref, p_ref, W1_ref, b1_ref, W2_ref, b2_ref,
                lnw_ref, lnb_ref, out_ref):
    x = x_ref[...]
    p = p_ref[...]
    h0 = x + p[0, :N] + p[1, :N]
    dn = (((1,), (1,)), ((), ()))
    h = lax.dot_general(h0, W1_ref[...], dn,
                        preferred_element_type=jnp.float32) + b1_ref[...]
    h = jnp.maximum(h, 0.0)
    h = lax.dot_general(h, W2_ref[...], dn,
                        preferred_element_type=jnp.float32) + b2_ref[...]
    h = h + x
    mean = jnp.mean(h)
    var = jnp.mean((h - mean) ** 2)
    h = (h - mean) / (jnp.sqrt(var) + 1e-5)
    h = h * lnw_ref[...] + lnb_ref[...]
    h = h * jax.nn.sigmoid(h)
    out_ref[...] = jnp.nan_to_num(h)


def _final_stage(x, partials, W1, b1, W2, b2, ln_w, ln_b):
    return pl.pallas_call(
        _final_body,
        out_shape=jax.ShapeDtypeStruct((N, D), jnp.float32),
    )(x, partials, W1, b1.reshape(1, D), W2, b2.reshape(1, D),
      ln_w.reshape(1, D), ln_b.reshape(1, D))


def kernel(x, edge_index, edge_attr, W_e, b_e, W1, b1, W2, b2, ln_w, ln_b):
    src = edge_index[0]
    dst = edge_index[1]
    npad = _EPAD - E
    pad_iota = jnp.arange(npad, dtype=jnp.int32)
    srcp = jnp.concatenate([src, pad_iota % N])
    dstp = jnp.concatenate([dst, N + pad_iota % (_NPAD - N)])
    e = _edge_mlp(edge_attr, W_e, b_e)
    partials = _message_passing(x, srcp, dstp, e)
    return _final_stage(x, partials, W1, b1, W2, b2, ln_w, ln_b)


# speedup vs baseline: 1.0191x; 1.0191x over previous
import functools

import jax
import jax.numpy as jnp
import numpy as np
from jax import lax
from jax.experimental import pallas as pl
from jax.experimental.pallas import tpu as pltpu
from jax.experimental.pallas import tpu_sc as plsc

N, E, D = 10000, 320000, 128

_NC, _NS, _L = 2, 16, 16
_NW = _NC * _NS
_CH = 64
_NCHUNKS = E // _CH
_CPW = 160
_CPAD = _NW * _CPW
_EPAD = _CPAD * _CH
_NBUF = 4
_NPAD = 10112
_RPT = _NPAD // _NS
_EBLK = 6400

_CLO = np.concatenate([np.arange(32 * g, 32 * g + 16) for g in range(D // 32)])
_CHI = _CLO + 16


def _edge_mlp_body(ea_ref, Wp_ref, bp_ref, out_ref):
    dn = (((1,), (1,)), ((), ()))
    e2 = lax.dot_general(ea_ref[...], Wp_ref[...], dn,
                         preferred_element_type=jnp.float32) + bp_ref[...]
    eb = lax.bitcast_convert_type(e2.astype(jnp.bfloat16),
                                  jnp.uint16).astype(jnp.uint32)
    out_ref[...] = eb[:, :D // 2] | lax.shift_left(eb[:, D // 2:],
                                                   jnp.uint32(16))


def _edge_mlp(edge_attr, W_e, b_e):
    cp = jnp.asarray(np.concatenate([_CLO, _CHI]))
    return pl.pallas_call(
        _edge_mlp_body,
        grid=(E // _EBLK,),
        in_specs=[
            pl.BlockSpec((_EBLK, D), lambda i: (i, 0)),
            pl.BlockSpec((D, D), lambda i: (0, 0)),
            pl.BlockSpec((1, D), lambda i: (0, 0)),
        ],
        out_specs=pl.BlockSpec((_EBLK, D // 2), lambda i: (i, 0)),
        out_shape=jax.ShapeDtypeStruct((E, D // 2), jnp.uint32),
    )(edge_attr, W_e[cp], b_e[cp].reshape(1, D))


_STRIPE = [(k * _CH, _CH) for k in range(_RPT // _CH)]
if _RPT % _CH:
    _STRIPE.append(((_RPT // _CH) * _CH, _RPT % _CH))

def _mp_body(x_hbm, src_hbm, dst_hbm, e_hbm, out_hbm,
             idx_s, idx_d, msgb, msgf, aggr_sh,
             sem_e, sem_g, sem_i, sem_sc):
    cid = lax.axis_index("c")
    sid = lax.axis_index("s")
    wid = cid * _NS + sid
    c0 = wid * _CPW

    def zrow(r, carry):
        for j in range(D // _L):
            msgf[0][r, pl.ds(j * _L, _L)] = jnp.zeros((_L,), jnp.float32)
        return carry
    lax.fori_loop(0, _CH, zrow, 0)
    for off, nr in _STRIPE:
        r0 = pl.multiple_of(sid * _RPT + off, 8)
        pltpu.sync_copy(msgf[0].at[pl.ds(0, nr)], aggr_sh.at[pl.ds(r0, nr)])
    plsc.subcore_barrier()

    def _ebase(c):
        g = c0 + c
        g = jnp.where(g < _NCHUNKS, g, 0)
        return pl.multiple_of(g * _CH, _CH)

    def _ibase(c):
        return pl.multiple_of((c0 + c) * _CH, _CH)

    def _i_issue(b, c):
        base = _ibase(c)
        pltpu.async_copy(src_hbm.at[pl.ds(base, _CH)], idx_s[b], sem_i[b])
        pltpu.async_copy(dst_hbm.at[pl.ds(base, _CH)], idx_d[b], sem_i[b])

    def _i_wait(b, c):
        base = _ibase(c)
        pltpu.make_async_copy(src_hbm.at[pl.ds(base, _CH)], idx_s[b],
                              sem_i[b]).wait()
        pltpu.make_async_copy(dst_hbm.at[pl.ds(base, _CH)], idx_d[b],
                              sem_i[b]).wait()

    def _e_issue(b, c):
        pltpu.async_copy(e_hbm.at[pl.ds(_ebase(c), _CH)], msgb[b], sem_e[b])

    def _e_wait(b, c):
        pltpu.make_async_copy(e_hbm.at[pl.ds(_ebase(c), _CH)], msgb[b],
                              sem_e[b]).wait()

    def _g_issue(b):
        pltpu.async_copy(x_hbm.at[idx_s[b]], msgf[b], sem_g[b])

    def _g_wait(b):
        pltpu.make_async_copy(x_hbm.at[idx_s[b]], msgf[b], sem_g[b]).wait()

    def _sc_issue(b):
        pltpu.async_copy(msgf[b], aggr_sh.at[idx_d[b]], sem_sc[b], add=True)

    def _sc_wait(b):
        pltpu.make_async_copy(msgf[b], aggr_sh.at[idx_d[b]], sem_sc[b]).wait()

    for k in range(2):
        _i_issue(k, k)
        _e_issue(k, k)
    for k in range(2):
        _i_wait(k, k)
        _g_issue(k)

    def body(t, carry):
        for b in range(_NBUF):
            c = t * _NBUF + b
            b2 = (b + 2) % _NBUF
            be = b % 2
            _g_wait(b)
            _e_wait(be, c)

            @pl.when(c >= 2)
            def _():
                _sc_wait(b2)

            @pl.when(c + 2 < _CPW)
            def _():
                _i_issue(b2, c + 2)

            hi_mask = jnp.uint32(0xFFFF0000)
            sh16 = jnp.uint32(16)

            def rrow(r, c2):
                for j in range(D // (2 * _L)):
                    u = msgb[be][r, pl.ds(j * _L, _L)]
                    lo = plsc.bitcast(lax.shift_left(u, sh16), jnp.float32)
                    hi = plsc.bitcast(u & hi_mask, jnp.float32)
                    jf = 2 * _L * j
                    x0 = msgf[b][r, pl.ds(jf, _L)]
                    x1 = msgf[b][r, pl.ds(jf + _L, _L)]
                    msgf[b][r, pl.ds(jf, _L)] = jnp.maximum(x0 + lo, 0.0)
                    msgf[b][r, pl.ds(jf + _L, _L)] = jnp.maximum(x1 + hi, 0.0)
                return c2
            lax.fori_loop(0, _CH, rrow, 0)

            @pl.when(c + 2 < _CPW)
            def _():
                _e_issue(be, c + 2)
                _i_wait(b2, c + 2)
                _g_issue(b2)
            _sc_issue(b)
        return carry
    lax.fori_loop(0, _CPW // _NBUF, body, 0)
    _sc_wait((_CPW - 2) % _NBUF)
    _sc_wait((_CPW - 1) % _NBUF)

    plsc.subcore_barrier()
    for off, nr in _STRIPE:
        r0 = pl.multiple_of(sid * _RPT + off, 8)
        pltpu.sync_copy(aggr_sh.at[pl.ds(r0, nr)], msgf[0].at[pl.ds(0, nr)])
        pltpu.sync_copy(msgf[0].at[pl.ds(0, nr)], out_hbm.at[cid, pl.ds(r0, nr)])


def _message_passing(x, src, dst, e):
    mesh = plsc.VectorSubcoreMesh(core_axis_name="c", subcore_axis_name="s")
    f = functools.partial(
        pl.kernel,
        out_type=jax.ShapeDtypeStruct((_NC, _NPAD, D), jnp.float32),
        mesh=mesh,
        compiler_params=pltpu.CompilerParams(needs_layout_passes=False),
        scratch_types=[
            [pltpu.VMEM((_CH,), jnp.int32)] * _NBUF,
            [pltpu.VMEM((_CH,), jnp.int32)] * _NBUF,
            [pltpu.VMEM((_CH, D // 2), jnp.uint32)] * 2,
            [pltpu.VMEM((_CH, D), jnp.float32)] * _NBUF,
            pltpu.VMEM_SHARED((_NPAD, D), jnp.float32),
            [pltpu.SemaphoreType.DMA] * 2,
            [pltpu.SemaphoreType.DMA] * _NBUF,
            [pltpu.SemaphoreType.DMA] * _NBUF,
            [pltpu.SemaphoreType.DMA] * _NBUF,
        ],
    )(_mp_body)
    return f(x, src, dst, e)


def _final_body(x_ref, p_ref, W1_ref, b1_ref, W2_ref, b2_ref,
                lnw_ref, lnb_ref, out_ref):
    x = x_ref[...]
    p = p_ref[...]
    h0 = x + p[0, :N] + p[1, :N]
    dn = (((1,), (1,)), ((), ()))
    h = lax.dot_general(h0, W1_ref[...], dn,
                        preferred_element_type=jnp.float32) + b1_ref[...]
    h = jnp.maximum(h, 0.0)
    h = lax.dot_general(h, W2_ref[...], dn,
                        preferred_element_type=jnp.float32) + b2_ref[...]
    h = h + x
    mean = jnp.mean(h)
    var = jnp.mean((h - mean) ** 2)
    h = (h - mean) / (jnp.sqrt(var) + 1e-5)
    h = h * lnw_ref[...] + lnb_ref[...]
    h = h * jax.nn.sigmoid(h)
    out_ref[...] = jnp.nan_to_num(h)


def _final_stage(x, partials, W1, b1, W2, b2, ln_w, ln_b):
    return pl.pallas_call(
        _final_body,
        out_shape=jax.ShapeDtypeStruct((N, D), jnp.float32),
    )(x, partials, W1, b1.reshape(1, D), W2, b2.reshape(1, D),
      ln_w.reshape(1, D), ln_b.reshape(1, D))


def kernel(x, edge_index, edge_attr, W_e, b_e, W1, b1, W2, b2, ln_w, ln_b):
    src = edge_index[0]
    dst = edge_index[1]
    npad = _EPAD - E
    pad_iota = jnp.arange(npad, dtype=jnp.int32)
    srcp = jnp.concatenate([src, pad_iota % N])
    dstp = jnp.concatenate([dst, N + pad_iota % (_NPAD - N)])
    e = _edge_mlp(edge_attr, W_e, b_e)
    partials = _message_passing(x, srcp, dstp, e)
    return _final_stage(x, partials, W1, b1, W2, b2, ln_w, ln_b)


# speedup vs baseline: 1.0424x; 1.0229x over previous
import functools

import jax
import jax.numpy as jnp
import numpy as np
from jax import lax
from jax.experimental import pallas as pl
from jax.experimental.pallas import tpu as pltpu
from jax.experimental.pallas import tpu_sc as plsc

N, E, D = 10000, 320000, 128

_NC, _NS, _L = 2, 16, 16
_NW = _NC * _NS
_CH = 64
_NCHUNKS = E // _CH
_CPW = 160
_CPAD = _NW * _CPW
_EPAD = _CPAD * _CH
_NBUF = 4
_NPAD = 10112
_RPT = _NPAD // _NS
_EBLK = 12800

_CLO = np.concatenate([np.arange(32 * g, 32 * g + 16) for g in range(D // 32)])
_CHI = _CLO + 16


def _edge_mlp_body(ea_ref, Wp_ref, bp_ref, out_ref):
    dn = (((1,), (1,)), ((), ()))
    e2 = lax.dot_general(ea_ref[...], Wp_ref[...], dn,
                         preferred_element_type=jnp.float32) + bp_ref[...]
    eb = lax.bitcast_convert_type(e2.astype(jnp.bfloat16),
                                  jnp.uint16).astype(jnp.uint32)
    out_ref[...] = eb[:, :D // 2] | lax.shift_left(eb[:, D // 2:],
                                                   jnp.uint32(16))


def _edge_mlp(edge_attr, W_e, b_e):
    cp = jnp.asarray(np.concatenate([_CLO, _CHI]))
    return pl.pallas_call(
        _edge_mlp_body,
        grid=(E // _EBLK,),
        in_specs=[
            pl.BlockSpec((_EBLK, D), lambda i: (i, 0)),
            pl.BlockSpec((D, D), lambda i: (0, 0)),
            pl.BlockSpec((1, D), lambda i: (0, 0)),
        ],
        out_specs=pl.BlockSpec((_EBLK, D // 2), lambda i: (i, 0)),
        out_shape=jax.ShapeDtypeStruct((E, D // 2), jnp.uint32),
    )(edge_attr, W_e[cp], b_e[cp].reshape(1, D))


_STRIPE = [(k * _CH, _CH) for k in range(_RPT // _CH)]
if _RPT % _CH:
    _STRIPE.append(((_RPT // _CH) * _CH, _RPT % _CH))

def _mp_body(x_hbm, src_hbm, dst_hbm, e_hbm, out_hbm,
             idx_s, idx_d, msgb, msgf, aggr_sh,
             sem_e, sem_g, sem_i, sem_sc):
    cid = lax.axis_index("c")
    sid = lax.axis_index("s")
    wid = cid * _NS + sid
    c0 = wid * _CPW

    def zrow(r, carry):
        for j in range(D // _L):
            msgf[0][r, pl.ds(j * _L, _L)] = jnp.zeros((_L,), jnp.float32)
        return carry
    lax.fori_loop(0, _CH, zrow, 0)
    for off, nr in _STRIPE:
        r0 = pl.multiple_of(sid * _RPT + off, 8)
        pltpu.sync_copy(msgf[0].at[pl.ds(0, nr)], aggr_sh.at[pl.ds(r0, nr)])
    plsc.subcore_barrier()

    def _ebase(c):
        g = c0 + c
        g = jnp.where(g < _NCHUNKS, g, 0)
        return pl.multiple_of(g * _CH, _CH)

    def _ibase(c):
        return pl.multiple_of((c0 + c) * _CH, _CH)

    def _i_issue(b, c):
        base = _ibase(c)
        pltpu.async_copy(src_hbm.at[pl.ds(base, _CH)], idx_s[b], sem_i[b])
        pltpu.async_copy(dst_hbm.at[pl.ds(base, _CH)], idx_d[b], sem_i[b])

    def _i_wait(b, c):
        base = _ibase(c)
        pltpu.make_async_copy(src_hbm.at[pl.ds(base, _CH)], idx_s[b],
                              sem_i[b]).wait()
        pltpu.make_async_copy(dst_hbm.at[pl.ds(base, _CH)], idx_d[b],
                              sem_i[b]).wait()

    def _e_issue(b, c):
        pltpu.async_copy(e_hbm.at[pl.ds(_ebase(c), _CH)], msgb[b], sem_e[b])

    def _e_wait(b, c):
        pltpu.make_async_copy(e_hbm.at[pl.ds(_ebase(c), _CH)], msgb[b],
                              sem_e[b]).wait()

    def _g_issue(b):
        pltpu.async_copy(x_hbm.at[idx_s[b]], msgf[b], sem_g[b])

    def _g_wait(b):
        pltpu.make_async_copy(x_hbm.at[idx_s[b]], msgf[b], sem_g[b]).wait()

    def _sc_issue(b):
        pltpu.async_copy(msgf[b], aggr_sh.at[idx_d[b]], sem_sc[b], add=True)

    def _sc_wait(b):
        pltpu.make_async_copy(msgf[b], aggr_sh.at[idx_d[b]], sem_sc[b]).wait()

    for k in range(2):
        _i_issue(k, k)
        _e_issue(k, k)
    for k in range(2):
        _i_wait(k, k)
        _g_issue(k)

    def body(t, carry):
        for b in range(_NBUF):
            c = t * _NBUF + b
            b2 = (b + 2) % _NBUF
            be = b % 2
            _g_wait(b)
            _e_wait(be, c)

            @pl.when(c >= 2)
            def _():
                _sc_wait(b2)

            @pl.when(c + 2 < _CPW)
            def _():
                _i_issue(b2, c + 2)

            hi_mask = jnp.uint32(0xFFFF0000)
            sh16 = jnp.uint32(16)

            def rrow(r, c2):
                for j in range(D // (2 * _L)):
                    u = msgb[be][r, pl.ds(j * _L, _L)]
                    lo = plsc.bitcast(lax.shift_left(u, sh16), jnp.float32)
                    hi = plsc.bitcast(u & hi_mask, jnp.float32)
                    jf = 2 * _L * j
                    x0 = msgf[b][r, pl.ds(jf, _L)]
                    x1 = msgf[b][r, pl.ds(jf + _L, _L)]
                    msgf[b][r, pl.ds(jf, _L)] = jnp.maximum(x0 + lo, 0.0)
                    msgf[b][r, pl.ds(jf + _L, _L)] = jnp.maximum(x1 + hi, 0.0)
                return c2
            lax.fori_loop(0, _CH, rrow, 0)

            @pl.when(c + 2 < _CPW)
            def _():
                _e_issue(be, c + 2)
                _i_wait(b2, c + 2)
                _g_issue(b2)
            _sc_issue(b)
        return carry
    lax.fori_loop(0, _CPW // _NBUF, body, 0)
    _sc_wait((_CPW - 2) % _NBUF)
    _sc_wait((_CPW - 1) % _NBUF)

    plsc.subcore_barrier()
    for off, nr in _STRIPE:
        r0 = pl.multiple_of(sid * _RPT + off, 8)
        pltpu.sync_copy(aggr_sh.at[pl.ds(r0, nr)], msgf[0].at[pl.ds(0, nr)])
        pltpu.sync_copy(msgf[0].at[pl.ds(0, nr)], out_hbm.at[cid, pl.ds(r0, nr)])


def _message_passing(x, src, dst, e):
    mesh = plsc.VectorSubcoreMesh(core_axis_name="c", subcore_axis_name="s")
    f = functools.partial(
        pl.kernel,
        out_type=jax.ShapeDtypeStruct((_NC, _NPAD, D), jnp.float32),
        mesh=mesh,
        compiler_params=pltpu.CompilerParams(needs_layout_passes=False),
        scratch_types=[
            [pltpu.VMEM((_CH,), jnp.int32)] * _NBUF,
            [pltpu.VMEM((_CH,), jnp.int32)] * _NBUF,
            [pltpu.VMEM((_CH, D // 2), jnp.uint32)] * 2,
            [pltpu.VMEM((_CH, D), jnp.float32)] * _NBUF,
            pltpu.VMEM_SHARED((_NPAD, D), jnp.float32),
            [pltpu.SemaphoreType.DMA] * 2,
            [pltpu.SemaphoreType.DMA] * _NBUF,
            [pltpu.SemaphoreType.DMA] * _NBUF,
            [pltpu.SemaphoreType.DMA] * _NBUF,
        ],
    )(_mp_body)
    return f(x, src, dst, e)


def _final_body(x_ref, p_ref, W1_ref, b1_ref, W2_ref, b2_ref,
                lnw_ref, lnb_ref, out_ref):
    x = x_ref[...]
    p = p_ref[...]
    h0 = x + p[0, :N] + p[1, :N]
    dn = (((1,), (1,)), ((), ()))
    h = lax.dot_general(h0, W1_ref[...], dn,
                        preferred_element_type=jnp.float32) + b1_ref[...]
    h = jnp.maximum(h, 0.0)
    h = lax.dot_general(h, W2_ref[...], dn,
                        preferred_element_type=jnp.float32) + b2_ref[...]
    h = h + x
    mean = jnp.mean(h)
    var = jnp.mean((h - mean) ** 2)
    h = (h - mean) / (jnp.sqrt(var) + 1e-5)
    h = h * lnw_ref[...] + lnb_ref[...]
    h = h * jax.nn.sigmoid(h)
    out_ref[...] = jnp.nan_to_num(h)


def _final_stage(x, partials, W1, b1, W2, b2, ln_w, ln_b):
    return pl.pallas_call(
        _final_body,
        out_shape=jax.ShapeDtypeStruct((N, D), jnp.float32),
    )(x, partials, W1, b1.reshape(1, D), W2, b2.reshape(1, D),
      ln_w.reshape(1, D), ln_b.reshape(1, D))


def kernel(x, edge_index, edge_attr, W_e, b_e, W1, b1, W2, b2, ln_w, ln_b):
    src = edge_index[0]
    dst = edge_index[1]
    npad = _EPAD - E
    pad_iota = jnp.arange(npad, dtype=jnp.int32)
    srcp = jnp.concatenate([src, pad_iota % N])
    dstp = jnp.concatenate([dst, N + pad_iota % (_NPAD - N)])
    e = _edge_mlp(edge_attr, W_e, b_e)
    partials = _message_passing(x, srcp, dstp, e)
    return _final_stage(x, partials, W1, b1, W2, b2, ln_w, ln_b)


# speedup vs baseline: 1.0441x; 1.0017x over previous
import functools

import jax
import jax.numpy as jnp
import numpy as np
from jax import lax
from jax.experimental import pallas as pl
from jax.experimental.pallas import tpu as pltpu
from jax.experimental.pallas import tpu_sc as plsc

N, E, D = 10000, 320000, 128

_NC, _NS, _L = 2, 16, 16
_NW = _NC * _NS
_CH = 64
_NCHUNKS = E // _CH
_CPW = 160
_CPAD = _NW * _CPW
_EPAD = _CPAD * _CH
_NBUF = 4
_NPAD = 10112
_RPT = _NPAD // _NS
_EBLK = 20000

_CLO = np.concatenate([np.arange(32 * g, 32 * g + 16) for g in range(D // 32)])
_CHI = _CLO + 16


def _edge_mlp_body(ea_ref, Wp_ref, bp_ref, out_ref):
    dn = (((1,), (1,)), ((), ()))
    e2 = lax.dot_general(ea_ref[...], Wp_ref[...], dn,
                         preferred_element_type=jnp.float32) + bp_ref[...]
    eb = lax.bitcast_convert_type(e2.astype(jnp.bfloat16),
                                  jnp.uint16).astype(jnp.uint32)
    out_ref[...] = eb[:, :D // 2] | lax.shift_left(eb[:, D // 2:],
                                                   jnp.uint32(16))


def _edge_mlp(edge_attr, W_e, b_e):
    cp = jnp.asarray(np.concatenate([_CLO, _CHI]))
    return pl.pallas_call(
        _edge_mlp_body,
        grid=(E // _EBLK,),
        in_specs=[
            pl.BlockSpec((_EBLK, D), lambda i: (i, 0)),
            pl.BlockSpec((D, D), lambda i: (0, 0)),
            pl.BlockSpec((1, D), lambda i: (0, 0)),
        ],
        out_specs=pl.BlockSpec((_EBLK, D // 2), lambda i: (i, 0)),
        out_shape=jax.ShapeDtypeStruct((E, D // 2), jnp.uint32),
    )(edge_attr, W_e[cp], b_e[cp].reshape(1, D))


_STRIPE = [(k * _CH, _CH) for k in range(_RPT // _CH)]
if _RPT % _CH:
    _STRIPE.append(((_RPT // _CH) * _CH, _RPT % _CH))

def _mp_body(x_hbm, src_hbm, dst_hbm, e_hbm, out_hbm,
             idx_s, idx_d, msgb, msgf, aggr_sh,
             sem_e, sem_g, sem_i, sem_sc):
    cid = lax.axis_index("c")
    sid = lax.axis_index("s")
    wid = cid * _NS + sid
    c0 = wid * _CPW

    def zrow(r, carry):
        for j in range(D // _L):
            msgf[0][r, pl.ds(j * _L, _L)] = jnp.zeros((_L,), jnp.float32)
        return carry
    lax.fori_loop(0, _CH, zrow, 0)
    for off, nr in _STRIPE:
        r0 = pl.multiple_of(sid * _RPT + off, 8)
        pltpu.sync_copy(msgf[0].at[pl.ds(0, nr)], aggr_sh.at[pl.ds(r0, nr)])
    plsc.subcore_barrier()

    def _ebase(c):
        g = c0 + c
        g = jnp.where(g < _NCHUNKS, g, 0)
        return pl.multiple_of(g * _CH, _CH)

    def _ibase(c):
        return pl.multiple_of((c0 + c) * _CH, _CH)

    def _i_issue(b, c):
        base = _ibase(c)
        pltpu.async_copy(src_hbm.at[pl.ds(base, _CH)], idx_s[b], sem_i[b])
        pltpu.async_copy(dst_hbm.at[pl.ds(base, _CH)], idx_d[b], sem_i[b])

    def _i_wait(b, c):
        base = _ibase(c)
        pltpu.make_async_copy(src_hbm.at[pl.ds(base, _CH)], idx_s[b],
                              sem_i[b]).wait()
        pltpu.make_async_copy(dst_hbm.at[pl.ds(base, _CH)], idx_d[b],
                              sem_i[b]).wait()

    def _e_issue(b, c):
        pltpu.async_copy(e_hbm.at[pl.ds(_ebase(c), _CH)], msgb[b], sem_e[b])

    def _e_wait(b, c):
        pltpu.make_async_copy(e_hbm.at[pl.ds(_ebase(c), _CH)], msgb[b],
                              sem_e[b]).wait()

    def _g_issue(b):
        pltpu.async_copy(x_hbm.at[idx_s[b]], msgf[b], sem_g[b])

    def _g_wait(b):
        pltpu.make_async_copy(x_hbm.at[idx_s[b]], msgf[b], sem_g[b]).wait()

    def _sc_issue(b):
        pltpu.async_copy(msgf[b], aggr_sh.at[idx_d[b]], sem_sc[b], add=True)

    def _sc_wait(b):
        pltpu.make_async_copy(msgf[b], aggr_sh.at[idx_d[b]], sem_sc[b]).wait()

    for k in range(2):
        _i_issue(k, k)
        _e_issue(k, k)
    for k in range(2):
        _i_wait(k, k)
        _g_issue(k)

    def body(t, carry):
        for b in range(_NBUF):
            c = t * _NBUF + b
            b2 = (b + 2) % _NBUF
            be = b % 2
            _g_wait(b)
            _e_wait(be, c)

            @pl.when(c >= 2)
            def _():
                _sc_wait(b2)

            @pl.when(c + 2 < _CPW)
            def _():
                _i_issue(b2, c + 2)

            hi_mask = jnp.uint32(0xFFFF0000)
            sh16 = jnp.uint32(16)

            def rrow(r, c2):
                for j in range(D // (2 * _L)):
                    u = msgb[be][r, pl.ds(j * _L, _L)]
                    lo = plsc.bitcast(lax.shift_left(u, sh16), jnp.float32)
                    hi = plsc.bitcast(u & hi_mask, jnp.float32)
                    jf = 2 * _L * j
                    x0 = msgf[b][r, pl.ds(jf, _L)]
                    x1 = msgf[b][r, pl.ds(jf + _L, _L)]
                    msgf[b][r, pl.ds(jf, _L)] = jnp.maximum(x0 + lo, 0.0)
                    msgf[b][r, pl.ds(jf + _L, _L)] = jnp.maximum(x1 + hi, 0.0)
                return c2
            lax.fori_loop(0, _CH, rrow, 0)

            @pl.when(c + 2 < _CPW)
            def _():
                _e_issue(be, c + 2)
                _i_wait(b2, c + 2)
                _g_issue(b2)
            _sc_issue(b)
        return carry
    lax.fori_loop(0, _CPW // _NBUF, body, 0)
    _sc_wait((_CPW - 2) % _NBUF)
    _sc_wait((_CPW - 1) % _NBUF)

    plsc.subcore_barrier()
    for off, nr in _STRIPE:
        r0 = pl.multiple_of(sid * _RPT + off, 8)
        pltpu.sync_copy(aggr_sh.at[pl.ds(r0, nr)], msgf[0].at[pl.ds(0, nr)])
        pltpu.sync_copy(msgf[0].at[pl.ds(0, nr)], out_hbm.at[cid, pl.ds(r0, nr)])


def _message_passing(x, src, dst, e):
    mesh = plsc.VectorSubcoreMesh(core_axis_name="c", subcore_axis_name="s")
    f = functools.partial(
        pl.kernel,
        out_type=jax.ShapeDtypeStruct((_NC, _NPAD, D), jnp.float32),
        mesh=mesh,
        compiler_params=pltpu.CompilerParams(needs_layout_passes=False),
        scratch_types=[
            [pltpu.VMEM((_CH,), jnp.int32)] * _NBUF,
            [pltpu.VMEM((_CH,), jnp.int32)] * _NBUF,
            [pltpu.VMEM((_CH, D // 2), jnp.uint32)] * 2,
            [pltpu.VMEM((_CH, D), jnp.float32)] * _NBUF,
            pltpu.VMEM_SHARED((_NPAD, D), jnp.float32),
            [pltpu.SemaphoreType.DMA] * 2,
            [pltpu.SemaphoreType.DMA] * _NBUF,
            [pltpu.SemaphoreType.DMA] * _NBUF,
            [pltpu.SemaphoreType.DMA] * _NBUF,
        ],
    )(_mp_body)
    return f(x, src, dst, e)


def _final_body(x_ref, p_ref, W1_ref, b1_ref, W2_ref, b2_ref,
                lnw_ref, lnb_ref, out_ref):
    x = x_ref[...]
    p = p_ref[...]
    h0 = x + p[0, :N] + p[1, :N]
    dn = (((1,), (1,)), ((), ()))
    h = lax.dot_general(h0, W1_ref[...], dn,
                        preferred_element_type=jnp.float32) + b1_ref[...]
    h = jnp.maximum(h, 0.0)
    h = lax.dot_general(h, W2_ref[...], dn,
                        preferred_element_type=jnp.float32) + b2_ref[...]
    h = h + x
    mean = jnp.mean(h)
    var = jnp.mean((h - mean) ** 2)
    h = (h - mean) / (jnp.sqrt(var) + 1e-5)
    h = h * lnw_ref[...] + lnb_ref[...]
    h = h * jax.nn.sigmoid(h)
    out_ref[...] = jnp.nan_to_num(h)


def _final_stage(x, partials, W1, b1, W2, b2, ln_w, ln_b):
    return pl.pallas_call(
        _final_body,
        out_shape=jax.ShapeDtypeStruct((N, D), jnp.float32),
    )(x, partials, W1, b1.reshape(1, D), W2, b2.reshape(1, D),
      ln_w.reshape(1, D), ln_b.reshape(1, D))


def kernel(x, edge_index, edge_attr, W_e, b_e, W1, b1, W2, b2, ln_w, ln_b):
    src = edge_index[0]
    dst = edge_index[1]
    npad = _EPAD - E
    pad_iota = jnp.arange(npad, dtype=jnp.int32)
    srcp = jnp.concatenate([src, pad_iota % N])
    dstp = jnp.concatenate([dst, N + pad_iota % (_NPAD - N)])
    e = _edge_mlp(edge_attr, W_e, b_e)
    partials = _message_passing(x, srcp, dstp, e)
    return _final_stage(x, partials, W1, b1, W2, b2, ln_w, ln_b)
